# baseline (speedup 1.0000x reference)
import functools
import math

import jax
import jax.numpy as jnp
from jax import lax
from jax.experimental import pallas as pl
from jax.experimental.pallas import tpu as pltpu
from jax.experimental.pallas import tpu_sc as plsc

D_MODEL = 64
SCALE = math.sqrt(D_MODEL)

_NUM_CORES = 2
_NUM_SUBCORES = 16
_NUM_WORKERS = _NUM_CORES * _NUM_SUBCORES
_LANES = 16

_CHUNK = 128


def _make_lookup(batch: int):
    assert batch % (_NUM_WORKERS * _CHUNK) == 0
    bpw = batch // _NUM_WORKERS
    nchunks = bpw // _CHUNK

    mesh = plsc.VectorSubcoreMesh(core_axis_name="c", subcore_axis_name="s")

    @functools.partial(
        pl.kernel,
        mesh=mesh,
        compiler_params=pltpu.CompilerParams(use_tc_tiling_on_sc=False),
        out_type=jax.ShapeDtypeStruct((batch, D_MODEL), jnp.float32),
        scratch_types=[
            pltpu.VMEM((_CHUNK,), jnp.int32),
            pltpu.VMEM((_CHUNK, D_MODEL), jnp.float32),
            pltpu.SemaphoreType.DMA,
        ],
    )
    def lookup(idx_hbm, table_hbm, out_hbm, idx_v, rows_v, sem):
        wid = lax.axis_index("s") * _NUM_CORES + lax.axis_index("c")
        base = wid * bpw

        def chunk_body(g, carry):
            off = base + g * _CHUNK
            pltpu.sync_copy(idx_hbm.at[pl.ds(off, _CHUNK)], idx_v)
            pltpu.async_copy(table_hbm.at[idx_v], rows_v, sem).wait()

            def scale_body(i, c2):
                for c in range(D_MODEL // _LANES):
                    sl = pl.ds(c * _LANES, _LANES)
                    rows_v[i, sl] = rows_v[i, sl] * SCALE
                return c2

            lax.fori_loop(0, _CHUNK, scale_body, 0)
            pltpu.sync_copy(rows_v, out_hbm.at[pl.ds(off, _CHUNK)])
            return carry

        lax.fori_loop(0, nchunks, chunk_body, 0)

    return lookup


def kernel(x, weight):
    batch = x.size
    xf = x.reshape(batch).astype(jnp.int32)
    out = _make_lookup(batch)(xf, weight)
    return out.reshape(*x.shape, D_MODEL)


# speedup vs baseline: 1.2738x; 1.2738x over previous
import functools
import math

import jax
import jax.numpy as jnp
from jax import lax
from jax.experimental import pallas as pl
from jax.experimental.pallas import tpu as pltpu
from jax.experimental.pallas import tpu_sc as plsc

D_MODEL = 64
SCALE = math.sqrt(D_MODEL)

_NUM_CORES = 2
_NUM_SUBCORES = 16
_NUM_WORKERS = _NUM_CORES * _NUM_SUBCORES
_LANES = 16

_CHUNK = 128
_NBUF = 4


def _make_lookup(batch: int):
    assert batch % (_NUM_WORKERS * _CHUNK) == 0
    bpw = batch // _NUM_WORKERS
    nchunk = bpw // _CHUNK
    assert nchunk % _NBUF == 0

    mesh = plsc.VectorSubcoreMesh(core_axis_name="c", subcore_axis_name="s")

    @functools.partial(
        pl.kernel,
        mesh=mesh,
        compiler_params=pltpu.CompilerParams(use_tc_tiling_on_sc=False),
        out_type=jax.ShapeDtypeStruct((batch, D_MODEL), jnp.float32),
        scratch_types=[
            pltpu.VMEM((nchunk, _CHUNK), jnp.int32),
            pltpu.VMEM((_NBUF, _CHUNK, D_MODEL), jnp.float32),
            pltpu.VMEM((_NBUF, _CHUNK, D_MODEL), jnp.float32),
            [pltpu.SemaphoreType.DMA] * _NBUF,
            [pltpu.SemaphoreType.DMA] * _NBUF,
        ],
    )
    def lookup(idx_hbm, table_hbm, out_hbm, idx_v, rows_g, rows_s, gsems, ssems):
        wid = lax.axis_index("s") * _NUM_CORES + lax.axis_index("c")
        row0 = wid * nchunk
        base = wid * bpw

        pltpu.sync_copy(idx_hbm.at[pl.ds(row0, nchunk)], idx_v)

        def fire_gather(c, b):
            pltpu.async_copy(table_hbm.at[idx_v.at[c]], rows_g.at[b], gsems[b])

        for c in range(_NBUF - 1):
            fire_gather(c, c)

        @pl.loop(0, nchunk, step=_NBUF)
        def _(go):
            for b in range(_NBUF):
                c = go + b
                bb = (b + _NBUF - 1) % _NBUF

                @pl.when(c + _NBUF - 1 < nchunk)
                def _():
                    fire_gather(c + _NBUF - 1, bb)

                pltpu.make_async_copy(
                    table_hbm.at[idx_v.at[c]], rows_g.at[b], gsems[b]).wait()

                @pl.when(go >= _NBUF)
                def _():
                    pltpu.make_async_copy(
                        rows_s.at[b],
                        out_hbm.at[pl.ds(base + (c - _NBUF) * _CHUNK, _CHUNK)],
                        ssems[b]).wait()

                @plsc.parallel_loop(0, _CHUNK, unroll=4)
                def _(i):
                    for q in range(D_MODEL // _LANES):
                        sl = pl.ds(q * _LANES, _LANES)
                        rows_s[b, i, sl] = rows_g[b, i, sl] * SCALE

                pltpu.async_copy(
                    rows_s.at[b],
                    out_hbm.at[pl.ds(base + c * _CHUNK, _CHUNK)],
                    ssems[b])

        for b in range(_NBUF):
            c = nchunk - _NBUF + b
            pltpu.make_async_copy(
                rows_s.at[b],
                out_hbm.at[pl.ds(base + c * _CHUNK, _CHUNK)],
                ssems[b]).wait()

    return lookup


def kernel(x, weight):
    batch = x.size
    xf = x.reshape(batch // _CHUNK, _CHUNK).astype(jnp.int32)
    out = _make_lookup(batch)(xf, weight)
    return out.reshape(*x.shape, D_MODEL)
